# baseline (speedup 1.0000x reference)
import functools

import jax
import jax.numpy as jnp
from jax import lax
from jax.experimental import pallas as pl
from jax.experimental.pallas import tpu as pltpu
from jax.experimental.pallas import tpu_sc as plsc

N_NODES = 10000
N_EDGES = 320000

NC, NS, LANES = 2, 16, 16
NW = NC * NS
CHUNK = 128
EPAD = 327680
NCHUNKS = EPAD // CHUNK
NPAD = 10112
ROWS_PER_SUB = NPAD // NS

_sc_mesh = plsc.VectorSubcoreMesh(
    core_axis_name="c", subcore_axis_name="s", num_cores=NC, num_subcores=NS
)

_untiled = pltpu.CompilerParams(use_tc_tiling_on_sc=False)


DEG_CH = NCHUNKS // NW


@functools.partial(
    pl.kernel,
    out_type=(
        jax.ShapeDtypeStruct((NC * NPAD, LANES), jnp.float32),
        jax.ShapeDtypeStruct((NC * NPAD, LANES), jnp.float32),
    ),
    mesh=_sc_mesh,
    compiler_params=_untiled,
    scratch_types=[
        pltpu.VMEM((DEG_CH, CHUNK), jnp.int32),
        pltpu.VMEM((DEG_CH, CHUNK), jnp.int32),
        pltpu.VMEM((CHUNK, LANES), jnp.float32),
        pltpu.VMEM_SHARED((NPAD, LANES), jnp.float32),
        pltpu.VMEM_SHARED((NPAD, LANES), jnp.float32),
    ],
)
def _sc_degrees(src_hbm, dst_hbm, ones_hbm, zeros_hbm, dego_out, degi_out,
                src_v, dst_v, ones_v, dego_sh, degi_sh):
    c = lax.axis_index("c")
    s = lax.axis_index("s")
    base = (c * NS + s) * DEG_CH
    pltpu.sync_copy(src_hbm.at[pl.ds(base, DEG_CH)], src_v)
    pltpu.sync_copy(dst_hbm.at[pl.ds(base, DEG_CH)], dst_v)
    pltpu.sync_copy(ones_hbm, ones_v)
    pltpu.sync_copy(zeros_hbm.at[pl.ds(s * ROWS_PER_SUB, ROWS_PER_SUB)],
                    dego_sh.at[pl.ds(s * ROWS_PER_SUB, ROWS_PER_SUB)])
    pltpu.sync_copy(zeros_hbm.at[pl.ds(s * ROWS_PER_SUB, ROWS_PER_SUB)],
                    degi_sh.at[pl.ds(s * ROWS_PER_SUB, ROWS_PER_SUB)])
    plsc.subcore_barrier()

    @pl.loop(0, DEG_CH)
    def _(j):
        pltpu.sync_copy(ones_v, dego_sh.at[src_v.at[j]], add=True)
        pltpu.sync_copy(ones_v, degi_sh.at[dst_v.at[j]], add=True)

    plsc.subcore_barrier()
    out_base = c * NPAD + s * ROWS_PER_SUB
    pltpu.sync_copy(dego_sh.at[pl.ds(s * ROWS_PER_SUB, ROWS_PER_SUB)],
                    dego_out.at[pl.ds(out_base, ROWS_PER_SUB)])
    pltpu.sync_copy(degi_sh.at[pl.ds(s * ROWS_PER_SUB, ROWS_PER_SUB)],
                    degi_out.at[pl.ds(out_base, ROWS_PER_SUB)])


def _make_sc_edge_pass(Fh):
    n_ch = NCHUNKS // NS

    @functools.partial(
        pl.kernel,
        out_type=jax.ShapeDtypeStruct((NC * NPAD, Fh), jnp.float32),
        mesh=_sc_mesh,
        compiler_params=_untiled,
        scratch_types=[
            pltpu.VMEM((n_ch, CHUNK), jnp.int32),
            pltpu.VMEM((n_ch, CHUNK), jnp.int32),
            pltpu.VMEM((CHUNK, Fh), jnp.float32),
            pltpu.VMEM((CHUNK, Fh), jnp.float32),
            pltpu.VMEM_SHARED((NPAD, Fh), jnp.float32),
            pltpu.SemaphoreType.DMA,
            pltpu.SemaphoreType.DMA,
            pltpu.SemaphoreType.DMA,
            pltpu.SemaphoreType.DMA,
        ],
    )
    def edge_pass(h_hbm, srcadj_hbm, dst_hbm, zeros_hbm, agg_out,
                  src_v, dst_v, rows_a, rows_b, agg_sh,
                  gsem_a, gsem_b, ssem_a, ssem_b):
        c = lax.axis_index("c")
        s = lax.axis_index("s")
        pltpu.sync_copy(srcadj_hbm.at[pl.ds(c * NCHUNKS + s * n_ch, n_ch)],
                        src_v)
        pltpu.sync_copy(dst_hbm.at[pl.ds(s * n_ch, n_ch)], dst_v)
        pltpu.sync_copy(zeros_hbm.at[pl.ds(s * ROWS_PER_SUB, ROWS_PER_SUB)],
                        agg_sh.at[pl.ds(s * ROWS_PER_SUB, ROWS_PER_SUB)])
        plsc.subcore_barrier()

        def gather(j, buf, sem):
            return pltpu.make_async_copy(h_hbm.at[src_v.at[j]], buf, sem)

        def scat(j, buf, sem):
            return pltpu.make_async_copy(buf, agg_sh.at[dst_v.at[j]], sem)

        def stage(j, buf, gsem, ssem):
            gather(j, buf, gsem).wait()
            scat(j, buf, ssem).start(add=True)

        gather(0, rows_a, gsem_a).start()

        @pl.loop(0, n_ch // 2 - 1)
        def _(i):
            j = 2 * i
            stage(j, rows_a, gsem_a, ssem_a)
            gather(j + 1, rows_b, gsem_b).start()
            stage(j + 1, rows_b, gsem_b, ssem_b)
            scat(j, rows_a, ssem_a).wait()
            gather(j + 2, rows_a, gsem_a).start()
            scat(j + 1, rows_b, ssem_b).wait()

        j = n_ch - 2
        stage(j, rows_a, gsem_a, ssem_a)
        gather(j + 1, rows_b, gsem_b).start()
        stage(j + 1, rows_b, gsem_b, ssem_b)
        scat(j, rows_a, ssem_a).wait()
        scat(j + 1, rows_b, ssem_b).wait()

        plsc.subcore_barrier()
        out_base = c * NPAD + s * ROWS_PER_SUB
        pltpu.sync_copy(agg_sh.at[pl.ds(s * ROWS_PER_SUB, ROWS_PER_SUB)],
                        agg_out.at[pl.ds(out_base, ROWS_PER_SUB)])

    return edge_pass


_sc_edge_pass_64 = _make_sc_edge_pass(64)
_sc_edge_pass_32 = _make_sc_edge_pass(32)


GRID = 8
BLK = NPAD // GRID


def _rsqrt_col(parts):
    d = parts[0] + parts[1]
    return lax.rsqrt(jnp.maximum(d[:, :1], 1.0))


def _tc_layer1(x_ref, w_ref, dego_ref, o_ref):
    scale = _rsqrt_col(dego_ref[...])
    h = jnp.dot(x_ref[...], w_ref[...], preferred_element_type=jnp.float32)
    h = h * scale
    o_ref[0] = h[:, :64]
    o_ref[1] = h[:, 64:]


def _tc_mid(agg_ref, degi_ref, dego_ref, b1_ref, w_ref, o_ref):
    a = jnp.concatenate([agg_ref[0], agg_ref[1]], axis=1)
    rin = _rsqrt_col(degi_ref[...])
    rout = _rsqrt_col(dego_ref[...])
    h = jnp.maximum(a * rin + b1_ref[...], 0.0)
    h2 = jnp.dot(h, w_ref[...], preferred_element_type=jnp.float32) * rout
    o_ref[0] = h2[:, :32]
    o_ref[1] = h2[:, 32:]


def _tc_final(agg_ref, degi_ref, b2_ref, o_ref):
    a = jnp.concatenate([agg_ref[0], agg_ref[1]], axis=1)
    rin = _rsqrt_col(degi_ref[...])
    o_ref[...] = a * rin + b2_ref[...]


def _row_spec(width):
    return pl.BlockSpec((BLK, width), lambda i: (i, 0))


def _parts_spec(width):
    return pl.BlockSpec((NC, BLK, width), lambda i: (0, i, 0))


def _full_spec(r, cw):
    return pl.BlockSpec((r, cw), lambda i: (0, 0))


def kernel(x, edge_index, W1, b1, W2, b2):
    f32 = jnp.float32
    src = edge_index[0].astype(jnp.int32)
    dst = edge_index[1].astype(jnp.int32)
    pad = N_NODES + (jnp.arange(EPAD - N_EDGES, dtype=jnp.int32)
                     % (NPAD - N_NODES))
    src2d = jnp.concatenate([src, pad]).reshape(NCHUNKS, CHUNK)
    dst2d = jnp.concatenate([dst, pad]).reshape(NCHUNKS, CHUNK)
    srcadj = jnp.concatenate([src2d, src2d + NPAD])

    xp = jnp.zeros((NPAD, 128), f32).at[:N_NODES].set(x)
    ones16 = jnp.ones((CHUNK, LANES), f32)
    zeros16 = jnp.zeros((NPAD, LANES), f32)
    zeros64 = jnp.zeros((NPAD, 64), f32)
    zeros32 = jnp.zeros((NPAD, 32), f32)

    dego_p, degi_p = _sc_degrees(src2d, dst2d, ones16, zeros16)
    dego_p = dego_p.reshape(NC, NPAD, LANES)
    degi_p = degi_p.reshape(NC, NPAD, LANES)

    h1 = pl.pallas_call(
        _tc_layer1,
        grid=(GRID,),
        in_specs=[_row_spec(128), _full_spec(128, 128), _parts_spec(LANES)],
        out_specs=_parts_spec(64),
        out_shape=jax.ShapeDtypeStruct((NC, NPAD, 64), f32),
    )(xp, W1, dego_p)

    agg1 = _sc_edge_pass_64(h1.reshape(NC * NPAD, 64), srcadj, dst2d,
                            zeros64).reshape(NC, NPAD, 64)

    h2 = pl.pallas_call(
        _tc_mid,
        grid=(GRID,),
        in_specs=[_parts_spec(64), _parts_spec(LANES), _parts_spec(LANES),
                  _full_spec(1, 128), _full_spec(128, 64)],
        out_specs=_parts_spec(32),
        out_shape=jax.ShapeDtypeStruct((NC, NPAD, 32), f32),
    )(agg1, degi_p, dego_p, b1.reshape(1, 128), W2)

    agg2 = _sc_edge_pass_32(h2.reshape(NC * NPAD, 32), srcadj, dst2d,
                            zeros32).reshape(NC, NPAD, 32)

    out = pl.pallas_call(
        _tc_final,
        grid=(GRID,),
        in_specs=[_parts_spec(32), _parts_spec(LANES), _full_spec(1, 64)],
        out_specs=_row_spec(64),
        out_shape=jax.ShapeDtypeStruct((NPAD, 64), f32),
    )(agg2, degi_p, b2.reshape(1, 64))

    return out[:N_NODES]


# speedup vs baseline: 1.1336x; 1.1336x over previous
import functools

import jax
import jax.numpy as jnp
from jax import lax
from jax.experimental import pallas as pl
from jax.experimental.pallas import tpu as pltpu
from jax.experimental.pallas import tpu_sc as plsc

N_NODES = 10000
N_EDGES = 320000

NC, NS, LANES = 2, 16, 16
NW = NC * NS
CHUNK = 128
EPAD = 327680
NCHUNKS = EPAD // CHUNK
NPAD = 10112
ROWS_PER_SUB = NPAD // NS

_sc_mesh = plsc.VectorSubcoreMesh(
    core_axis_name="c", subcore_axis_name="s", num_cores=NC, num_subcores=NS
)

_untiled = pltpu.CompilerParams(use_tc_tiling_on_sc=False)


DEG_CH = NCHUNKS // NW


@functools.partial(
    pl.kernel,
    out_type=(
        jax.ShapeDtypeStruct((NC * NPAD, LANES), jnp.float32),
        jax.ShapeDtypeStruct((NC * NPAD, LANES), jnp.float32),
    ),
    mesh=_sc_mesh,
    compiler_params=_untiled,
    scratch_types=[
        pltpu.VMEM((DEG_CH, CHUNK), jnp.int32),
        pltpu.VMEM((DEG_CH, CHUNK), jnp.int32),
        pltpu.VMEM((CHUNK, LANES), jnp.float32),
        pltpu.VMEM_SHARED((NPAD, LANES), jnp.float32),
        pltpu.VMEM_SHARED((NPAD, LANES), jnp.float32),
        pltpu.SemaphoreType.DMA,
        pltpu.SemaphoreType.DMA,
    ],
)
def _sc_degrees(src_hbm, dst_hbm, ones_hbm, zeros_hbm, dego_out, degi_out,
                src_v, dst_v, ones_v, dego_sh, degi_sh, dsem, isem):
    c = lax.axis_index("c")
    s = lax.axis_index("s")
    base = (c * NS + s) * DEG_CH
    pltpu.sync_copy(src_hbm.at[pl.ds(base, DEG_CH)], src_v)
    pltpu.sync_copy(dst_hbm.at[pl.ds(base, DEG_CH)], dst_v)
    pltpu.sync_copy(ones_hbm, ones_v)
    pltpu.sync_copy(zeros_hbm.at[pl.ds(s * ROWS_PER_SUB, ROWS_PER_SUB)],
                    dego_sh.at[pl.ds(s * ROWS_PER_SUB, ROWS_PER_SUB)])
    pltpu.sync_copy(zeros_hbm.at[pl.ds(s * ROWS_PER_SUB, ROWS_PER_SUB)],
                    degi_sh.at[pl.ds(s * ROWS_PER_SUB, ROWS_PER_SUB)])
    plsc.subcore_barrier()

    @pl.loop(0, DEG_CH)
    def _(j):
        o_cp = pltpu.make_async_copy(ones_v, dego_sh.at[src_v.at[j]], dsem)
        i_cp = pltpu.make_async_copy(ones_v, degi_sh.at[dst_v.at[j]], isem)
        o_cp.start(add=True)
        i_cp.start(add=True)
        o_cp.wait()
        i_cp.wait()

    plsc.subcore_barrier()
    out_base = c * NPAD + s * ROWS_PER_SUB
    pltpu.sync_copy(dego_sh.at[pl.ds(s * ROWS_PER_SUB, ROWS_PER_SUB)],
                    dego_out.at[pl.ds(out_base, ROWS_PER_SUB)])
    pltpu.sync_copy(degi_sh.at[pl.ds(s * ROWS_PER_SUB, ROWS_PER_SUB)],
                    degi_out.at[pl.ds(out_base, ROWS_PER_SUB)])


def _make_sc_edge_pass(Fh, feature_split):
    n_ch = NCHUNKS // NS if feature_split else NCHUNKS // NW

    @functools.partial(
        pl.kernel,
        out_type=jax.ShapeDtypeStruct((NC * NPAD, Fh), jnp.float32),
        mesh=_sc_mesh,
        compiler_params=_untiled,
        scratch_types=[
            pltpu.VMEM((n_ch, CHUNK), jnp.int32),
            pltpu.VMEM((n_ch, CHUNK), jnp.int32),
            pltpu.VMEM((CHUNK, Fh), jnp.float32),
            pltpu.VMEM((CHUNK, Fh), jnp.float32),
            pltpu.VMEM_SHARED((NPAD, Fh), jnp.float32),
            pltpu.SemaphoreType.DMA,
            pltpu.SemaphoreType.DMA,
            pltpu.SemaphoreType.DMA,
            pltpu.SemaphoreType.DMA,
        ],
    )
    def edge_pass(h_hbm, srcadj_hbm, dst_hbm, zeros_hbm, agg_out,
                  src_v, dst_v, rows_a, rows_b, agg_sh,
                  gsem_a, gsem_b, ssem_a, ssem_b):
        c = lax.axis_index("c")
        s = lax.axis_index("s")
        if feature_split:
            src_base = c * NCHUNKS + s * n_ch
            dst_base = s * n_ch
        else:
            src_base = dst_base = (c * NS + s) * n_ch
        pltpu.sync_copy(srcadj_hbm.at[pl.ds(src_base, n_ch)], src_v)
        pltpu.sync_copy(dst_hbm.at[pl.ds(dst_base, n_ch)], dst_v)
        pltpu.sync_copy(zeros_hbm.at[pl.ds(s * ROWS_PER_SUB, ROWS_PER_SUB)],
                        agg_sh.at[pl.ds(s * ROWS_PER_SUB, ROWS_PER_SUB)])
        plsc.subcore_barrier()

        def gather(j, buf, sem):
            return pltpu.make_async_copy(h_hbm.at[src_v.at[j]], buf, sem)

        def scat(j, buf, sem):
            return pltpu.make_async_copy(buf, agg_sh.at[dst_v.at[j]], sem)

        def stage(j, buf, gsem, ssem):
            gather(j, buf, gsem).wait()
            scat(j, buf, ssem).start(add=True)

        gather(0, rows_a, gsem_a).start()

        @pl.loop(0, n_ch // 2 - 1)
        def _(i):
            j = 2 * i
            stage(j, rows_a, gsem_a, ssem_a)
            gather(j + 1, rows_b, gsem_b).start()
            stage(j + 1, rows_b, gsem_b, ssem_b)
            scat(j, rows_a, ssem_a).wait()
            gather(j + 2, rows_a, gsem_a).start()
            scat(j + 1, rows_b, ssem_b).wait()

        j = n_ch - 2
        stage(j, rows_a, gsem_a, ssem_a)
        gather(j + 1, rows_b, gsem_b).start()
        stage(j + 1, rows_b, gsem_b, ssem_b)
        scat(j, rows_a, ssem_a).wait()
        scat(j + 1, rows_b, ssem_b).wait()

        plsc.subcore_barrier()
        out_base = c * NPAD + s * ROWS_PER_SUB
        pltpu.sync_copy(agg_sh.at[pl.ds(s * ROWS_PER_SUB, ROWS_PER_SUB)],
                        agg_out.at[pl.ds(out_base, ROWS_PER_SUB)])

    return edge_pass


_sc_edge_pass_l1 = _make_sc_edge_pass(64, True)
_sc_edge_pass_l2 = _make_sc_edge_pass(64, False)


GRID = 8
BLK = NPAD // GRID


def _rsqrt_col(parts):
    d = parts[0] + parts[1]
    return lax.rsqrt(jnp.maximum(d[:, :1], 1.0))


def _tc_layer1(x_ref, w_ref, dego_ref, o_ref):
    scale = _rsqrt_col(dego_ref[...])
    h = jnp.dot(x_ref[...], w_ref[...], preferred_element_type=jnp.float32)
    h = h * scale
    o_ref[0] = h[:, :64]
    o_ref[1] = h[:, 64:]


def _tc_mid(agg_ref, degi_ref, dego_ref, b1_ref, w_ref, o_ref):
    a = jnp.concatenate([agg_ref[0], agg_ref[1]], axis=1)
    rin = _rsqrt_col(degi_ref[...])
    rout = _rsqrt_col(dego_ref[...])
    h = jnp.maximum(a * rin + b1_ref[...], 0.0)
    o_ref[...] = jnp.dot(h, w_ref[...],
                         preferred_element_type=jnp.float32) * rout


def _tc_final(agg_ref, degi_ref, b2_ref, o_ref):
    a = agg_ref[0] + agg_ref[1]
    rin = _rsqrt_col(degi_ref[...])
    o_ref[...] = a * rin + b2_ref[...]


def _row_spec(width):
    return pl.BlockSpec((BLK, width), lambda i: (i, 0))


def _parts_spec(width):
    return pl.BlockSpec((NC, BLK, width), lambda i: (0, i, 0))


def _full_spec(r, cw):
    return pl.BlockSpec((r, cw), lambda i: (0, 0))


def kernel(x, edge_index, W1, b1, W2, b2):
    f32 = jnp.float32
    src = edge_index[0].astype(jnp.int32)
    dst = edge_index[1].astype(jnp.int32)
    pad = N_NODES + (jnp.arange(EPAD - N_EDGES, dtype=jnp.int32)
                     % (NPAD - N_NODES))
    src2d = jnp.concatenate([src, pad]).reshape(NCHUNKS, CHUNK)
    dst2d = jnp.concatenate([dst, pad]).reshape(NCHUNKS, CHUNK)
    srcadj = jnp.concatenate([src2d, src2d + NPAD])

    xp = jnp.zeros((NPAD, 128), f32).at[:N_NODES].set(x)
    ones16 = jnp.ones((CHUNK, LANES), f32)
    zeros16 = jnp.zeros((NPAD, LANES), f32)
    zeros64 = jnp.zeros((NPAD, 64), f32)

    dego_p, degi_p = _sc_degrees(src2d, dst2d, ones16, zeros16)
    dego_p = dego_p.reshape(NC, NPAD, LANES)
    degi_p = degi_p.reshape(NC, NPAD, LANES)

    h1 = pl.pallas_call(
        _tc_layer1,
        grid=(GRID,),
        in_specs=[_row_spec(128), _full_spec(128, 128), _parts_spec(LANES)],
        out_specs=_parts_spec(64),
        out_shape=jax.ShapeDtypeStruct((NC, NPAD, 64), f32),
    )(xp, W1, dego_p)

    agg1 = _sc_edge_pass_l1(h1.reshape(NC * NPAD, 64), srcadj, dst2d,
                            zeros64).reshape(NC, NPAD, 64)

    h2 = pl.pallas_call(
        _tc_mid,
        grid=(GRID,),
        in_specs=[_parts_spec(64), _parts_spec(LANES), _parts_spec(LANES),
                  _full_spec(1, 128), _full_spec(128, 64)],
        out_specs=_row_spec(64),
        out_shape=jax.ShapeDtypeStruct((NPAD, 64), f32),
    )(agg1, degi_p, dego_p, b1.reshape(1, 128), W2)

    agg2 = _sc_edge_pass_l2(h2, src2d, dst2d,
                            zeros64).reshape(NC, NPAD, 64)

    out = pl.pallas_call(
        _tc_final,
        grid=(GRID,),
        in_specs=[_parts_spec(64), _parts_spec(LANES), _full_spec(1, 64)],
        out_specs=_row_spec(64),
        out_shape=jax.ShapeDtypeStruct((NPAD, 64), f32),
    )(agg2, degi_p, b2.reshape(1, 64))

    return out[:N_NODES]
